# baseline (speedup 1.0000x reference)
import functools

import jax
import jax.numpy as jnp
from jax import lax
from jax.experimental import pallas as pl
from jax.experimental.pallas import tpu as pltpu
from jax.experimental.pallas import tpu_sc as plsc

VOCAB = 1_000_000
EMB = 64
BATCH = 4096
SEQ = 200

NUM_IDS = BATCH * SEQ
W32 = EMB // 2

_info = plsc.get_sparse_core_info()
NC, NS = _info.num_cores, _info.num_subcores
NW = NC * NS
PER_W = NUM_IDS // NW
IDX_ROW = 128
N_GATHERS = PER_W // IDX_ROW
GROUP = 8
ROWS_PER_GROUP = GROUP * IDX_ROW
N_GROUPS = N_GATHERS // GROUP


def _sc_gather(ids_flat, weight_i32):
    mesh = plsc.VectorSubcoreMesh(core_axis_name="c", subcore_axis_name="s")

    @functools.partial(
        pl.kernel,
        mesh=mesh,
        out_type=jax.ShapeDtypeStruct((NUM_IDS, W32), jnp.int32),
        scratch_types=[
            pltpu.VMEM((N_GATHERS, IDX_ROW), jnp.int32),
            pltpu.VMEM((ROWS_PER_GROUP, W32), jnp.int32),
            pltpu.SemaphoreType.DMA,
            pltpu.SemaphoreType.DMA,
        ],
        compiler_params=pltpu.CompilerParams(use_tc_tiling_on_sc=False),
    )
    def k(ids_hbm, w_hbm, out_hbm, idx_v, rows_v, sem_g, sem_w):
        wid = lax.axis_index("s") * NC + lax.axis_index("c")
        base = wid * PER_W
        pltpu.sync_copy(ids_hbm.at[pl.ds(wid * N_GATHERS, N_GATHERS)], idx_v)

        def body(t, _):
            for g in range(GROUP):
                j = t * GROUP + g
                pltpu.async_copy(
                    w_hbm.at[idx_v.at[j]],
                    rows_v.at[pl.ds(g * IDX_ROW, IDX_ROW)],
                    sem_g,
                )
            for g in range(GROUP):
                pltpu.make_async_copy(
                    w_hbm.at[idx_v.at[0]],
                    rows_v.at[pl.ds(0, IDX_ROW)],
                    sem_g,
                ).wait()
            pltpu.async_copy(
                rows_v,
                out_hbm.at[pl.ds(base + t * ROWS_PER_GROUP, ROWS_PER_GROUP)],
                sem_w,
            ).wait()
            return ()

        lax.fori_loop(0, N_GROUPS, body, ())

    ids2d = ids_flat.reshape(NW * N_GATHERS, IDX_ROW)
    return k(ids2d, weight_i32)


def _tc_transpose(emb2d):
    G = 16

    def body(e_ref, o_ref):
        for g in range(G):
            blk = e_ref[pl.ds(g * SEQ, SEQ), :]
            o_ref[pl.ds(g * EMB, EMB), :] = jnp.transpose(blk, (1, 0))

    out = pl.pallas_call(
        body,
        grid=(BATCH // G,),
        in_specs=[pl.BlockSpec((G * SEQ, EMB), lambda i: (i, 0))],
        out_specs=pl.BlockSpec((G * EMB, SEQ), lambda i: (i, 0)),
        out_shape=jax.ShapeDtypeStruct((BATCH * EMB, SEQ), jnp.bfloat16),
    )(emb2d)
    return out.reshape(BATCH, EMB, SEQ)


def kernel(ids, weight):
    weight_i32 = lax.bitcast_convert_type(
        weight.reshape(VOCAB, W32, 2), jnp.int32
    )
    ids_flat = ids.reshape(NUM_IDS)
    emb_i32 = _sc_gather(ids_flat, weight_i32)
    emb = lax.bitcast_convert_type(emb_i32, jnp.bfloat16).reshape(
        BATCH * SEQ, EMB
    )
    out_bf = _tc_transpose(emb)
    return lax.bitcast_convert_type(out_bf, jnp.float16)


# speedup vs baseline: 2.4740x; 2.4740x over previous
import functools

import jax
import jax.numpy as jnp
from jax import lax
from jax.experimental import pallas as pl
from jax.experimental.pallas import tpu as pltpu
from jax.experimental.pallas import tpu_sc as plsc

VOCAB = 1_000_000
EMB = 64
BATCH = 4096
SEQ = 200
S2 = SEQ // 2
OUT_W = EMB * S2

_info = plsc.get_sparse_core_info()
NC, NS = _info.num_cores, _info.num_subcores
NW = NC * NS
B_PER_W = BATCH // NW


def _fused_embed(ids, weight):
    mesh = plsc.VectorSubcoreMesh(core_axis_name="c", subcore_axis_name="s")

    @functools.partial(
        pl.kernel,
        mesh=mesh,
        out_type=jax.ShapeDtypeStruct((BATCH, OUT_W), jnp.int32),
        scratch_types=[
            pltpu.VMEM((B_PER_W, SEQ), jnp.int32),
            pltpu.VMEM((SEQ, EMB), jnp.float16),
            pltpu.VMEM((OUT_W,), jnp.int32),
            pltpu.SemaphoreType.DMA,
            pltpu.SemaphoreType.DMA,
        ],
        compiler_params=pltpu.CompilerParams(
            use_tc_tiling_on_sc=False, needs_layout_passes=False
        ),
    )
    def k(ids_hbm, w_hbm, out_hbm, ids_v, g_v, t_v, sem_g, sem_w):
        wid = lax.axis_index("s") * NC + lax.axis_index("c")
        b0 = wid * B_PER_W
        pltpu.sync_copy(ids_hbm.at[pl.ds(b0, B_PER_W)], ids_v)

        lane = lax.broadcasted_iota(jnp.int32, (16,), 0)
        p_even = lane * (2 * S2)
        lo_mask = jnp.full((16,), 0xFFFF, jnp.int32)
        hi_mask = jnp.full((16,), -0x10000, jnp.int32)

        def body(bl, _):
            cp0 = pltpu.async_copy(
                w_hbm.at[ids_v.at[bl, pl.ds(0, 128)]],
                g_v.at[pl.ds(0, 128)],
                sem_g,
            )
            cp1 = pltpu.async_copy(
                w_hbm.at[ids_v.at[bl, pl.ds(128, 72)]],
                g_v.at[pl.ds(128, 72)],
                sem_g,
            )
            cp0.wait()
            cp1.wait()

            def col(s2, _):
                a = plsc.bitcast(g_v[2 * s2, pl.ds(0, 32)], jnp.int32)
                b = plsc.bitcast(g_v[2 * s2 + 1, pl.ds(0, 32)], jnp.int32)
                a2 = plsc.bitcast(g_v[2 * s2, pl.ds(32, 32)], jnp.int32)
                b2 = plsc.bitcast(g_v[2 * s2 + 1, pl.ds(32, 32)], jnp.int32)
                c0 = (a & lo_mask) | lax.shift_left(b, 16)
                c1 = lax.shift_right_logical(a, 16) | (b & hi_mask)
                c2 = (a2 & lo_mask) | lax.shift_left(b2, 16)
                c3 = lax.shift_right_logical(a2, 16) | (b2 & hi_mask)
                plsc.store_scatter(t_v, [p_even + s2], c0)
                plsc.store_scatter(t_v, [p_even + (S2 + s2)], c1)
                plsc.store_scatter(t_v, [p_even + (32 * S2 + s2)], c2)
                plsc.store_scatter(t_v, [p_even + (33 * S2 + s2)], c3)
                return ()

            lax.fori_loop(0, S2, col, ())
            pltpu.async_copy(t_v, out_hbm.at[b0 + bl], sem_w).wait()
            return ()

        lax.fori_loop(0, B_PER_W, body, ())

    return k(ids, weight)


def kernel(ids, weight):
    out_i32 = _fused_embed(ids, weight)
    out = lax.bitcast_convert_type(out_i32, jnp.float16)
    return out.reshape(BATCH, EMB, SEQ)
